# baseline (speedup 1.0000x reference)
import functools

import jax
import jax.numpy as jnp
from jax.experimental import pallas as pl
from jax.experimental.pallas import tpu as pltpu
from jax.experimental.pallas import tpu_sc as plsc

E = 8
DM = 768
DF = 2048
T = 2048
BLK = 256
NPAD = 2 * T + E * BLK
NBLK = NPAD // BLK
FJ = 2
FB = DF // FJ
NS = 32
NW = 32
TPW = T // NW


def _router_body(h_ref, gwt_ref, logits_ref, dest_ref, wall_ref, sarr_ref):
    hb = h_ref[...].astype(jnp.bfloat16)
    logits = jax.lax.dot_general(
        hb, gwt_ref[...].astype(jnp.bfloat16), (((1,), (0,)), ((), ())),
        preferred_element_type=jnp.float32)
    logits_ref[...] = logits
    m = jnp.max(logits, axis=1, keepdims=True)
    ex = jnp.exp(logits - m)
    p = ex / jnp.sum(ex, axis=1, keepdims=True)
    idxs = jax.lax.broadcasted_iota(jnp.int32, (T, E), 1)
    m0 = jnp.max(p, axis=1, keepdims=True)
    i0 = jnp.min(jnp.where(p == m0, idxs, E), axis=1, keepdims=True)
    p1 = jnp.where(idxs == i0, -jnp.inf, p)
    m1 = jnp.max(p1, axis=1, keepdims=True)
    i1 = jnp.min(jnp.where(p1 == m1, idxs, E), axis=1, keepdims=True)
    s = m0 + m1
    wall_ref[...] = jnp.concatenate([m0 / s, m1 / s], axis=0)

    ohc = jnp.concatenate([idxs == i0, idxs == i1], axis=0)
    c = ohc.astype(jnp.float32)
    sh = 1
    while sh < 2 * T:
        c = c + jnp.concatenate(
            [jnp.zeros((sh, E), jnp.float32), c[:2 * T - sh]], axis=0)
        sh *= 2
    counts = c[2 * T - 1:2 * T, :].astype(jnp.int32)
    padded = ((counts + (BLK - 1)) // BLK) * BLK
    incl = padded
    sh = 1
    while sh < E:
        incl = incl + jnp.concatenate(
            [jnp.zeros((1, sh), jnp.int32), incl[:, :E - sh]], axis=1)
        sh *= 2
    off = incl - padded
    dest = jnp.sum(
        jnp.where(ohc, c + off.astype(jnp.float32) - 1.0, 0.0),
        axis=1, keepdims=True)
    dest_ref[...] = dest.astype(jnp.int32)

    ub = incl[0:1, E - 1:E] // BLK
    bs = jax.lax.broadcasted_iota(jnp.int32, (NS, 1), 0) * BLK
    be = jnp.sum((incl <= bs).astype(jnp.int32), axis=1, keepdims=True)
    eidx = jax.lax.broadcasted_iota(jnp.int32, (1, E), 1)
    last_e = jnp.max(jnp.where(padded > 0, eidx, 0), axis=1, keepdims=True)
    be = jnp.where(bs < ub * BLK, be, last_e)
    sarr_ref[...] = jnp.concatenate(
        [be, jnp.broadcast_to(ub, (8, 1))], axis=0)


def _router(hidden_states, gwt):
    return pl.pallas_call(
        _router_body,
        out_shape=[
            jax.ShapeDtypeStruct((T, E), jnp.float32),
            jax.ShapeDtypeStruct((2 * T, 1), jnp.int32),
            jax.ShapeDtypeStruct((2 * T, 1), jnp.float32),
            jax.ShapeDtypeStruct((NS + 8, 1), jnp.int32),
        ],
    )(hidden_states, gwt)


def _dispatch(hidden_states, dest, wall):
    mesh = plsc.VectorSubcoreMesh(core_axis_name="c", subcore_axis_name="s")

    @functools.partial(
        pl.kernel, mesh=mesh,
        out_type=[
            jax.ShapeDtypeStruct((NPAD, DM), jnp.float32),
            jax.ShapeDtypeStruct((NPAD,), jnp.float32),
        ],
        scratch_types=[
            pltpu.VMEM((TPW, DM), jnp.float32),
            pltpu.VMEM((TPW,), jnp.int32),
            pltpu.VMEM((TPW,), jnp.float32),
            pltpu.SemaphoreType.DMA,
        ],
    )
    def disp(h_hbm, d_hbm, w_hbm, x_out, w_out, rows_v, idx_v, wv, sem):
        wid = jax.lax.axis_index("s") * 2 + jax.lax.axis_index("c")
        base = wid * TPW
        pltpu.sync_copy(h_hbm.at[pl.ds(base, TPW)], rows_v)
        pltpu.sync_copy(d_hbm.at[pl.ds(base, TPW)], idx_v)
        pltpu.sync_copy(w_hbm.at[pl.ds(base, TPW)], wv)
        pltpu.async_copy(rows_v, x_out.at[idx_v], sem).wait()
        pltpu.async_copy(wv, w_out.at[idx_v], sem).wait()
        pltpu.sync_copy(d_hbm.at[pl.ds(T + base, TPW)], idx_v)
        pltpu.sync_copy(w_hbm.at[pl.ds(T + base, TPW)], wv)
        pltpu.async_copy(rows_v, x_out.at[idx_v], sem).wait()
        pltpu.async_copy(wv, w_out.at[idx_v], sem).wait()

    return disp(hidden_states, dest, wall)


def _gmm_body(s_ref, x_ref, wg_ref, wu_ref, bg_ref, bu_ref, wd_ref, bd_ref,
              wt_ref, y_ref, acc_ref):
    j = pl.program_id(1)

    @pl.when(pl.program_id(0) < s_ref[NS])
    def _():
        x = x_ref[...]
        g = jnp.dot(x, wg_ref[0], precision=jax.lax.Precision.DEFAULT,
                    preferred_element_type=jnp.float32) + bg_ref[0]
        u = jnp.dot(x, wu_ref[0], precision=jax.lax.Precision.DEFAULT,
                    preferred_element_type=jnp.float32) + bu_ref[0]
        inter = (g * jax.lax.logistic(g)) * u
        y = jnp.dot(inter, wd_ref[0], precision=jax.lax.Precision.DEFAULT,
                    preferred_element_type=jnp.float32)

        @pl.when(j == 0)
        def _():
            acc_ref[...] = y

        @pl.when(j > 0)
        def _():
            acc_ref[...] += y

        @pl.when(j == FJ - 1)
        def _():
            y_ref[...] = (acc_ref[...] + bd_ref[0]) * wt_ref[...]


def _frozen_j(i, j, s):
    return jnp.where(i < s[NS], j, FJ - 1)


def _gmm(s1, x_sorted, gate_up_proj, gub, down_proj, db, w_sorted):
    grid_spec = pltpu.PrefetchScalarGridSpec(
        num_scalar_prefetch=1,
        grid=(NBLK, FJ),
        in_specs=[
            pl.BlockSpec((BLK, DM),
                         lambda i, j, s: (jnp.minimum(i, s[NS] - 1), 0)),
            pl.BlockSpec((1, DM, FB), lambda i, j, s: (s[i], 0, _frozen_j(i, j, s))),
            pl.BlockSpec((1, DM, FB),
                         lambda i, j, s: (s[i], 0, _frozen_j(i, j, s) + FJ)),
            pl.BlockSpec((1, 1, FB), lambda i, j, s: (s[i], 0, _frozen_j(i, j, s))),
            pl.BlockSpec((1, 1, FB),
                         lambda i, j, s: (s[i], 0, _frozen_j(i, j, s) + FJ)),
            pl.BlockSpec((1, FB, DM), lambda i, j, s: (s[i], _frozen_j(i, j, s), 0)),
            pl.BlockSpec((1, 1, DM), lambda i, j, s: (s[i], 0, 0)),
            pl.BlockSpec((BLK, 1),
                         lambda i, j, s: (jnp.minimum(i, s[NS] - 1), 0)),
        ],
        out_specs=pl.BlockSpec((BLK, DM),
                               lambda i, j, s: (jnp.minimum(i, s[NS] - 1), 0)),
        scratch_shapes=[pltpu.VMEM((BLK, DM), jnp.float32)],
    )
    return pl.pallas_call(
        _gmm_body,
        grid_spec=grid_spec,
        out_shape=jax.ShapeDtypeStruct((NPAD, DM), jnp.float32),
    )(s1, x_sorted, gate_up_proj, gate_up_proj, gub, gub, down_proj, db,
      w_sorted)


def _combine(y_sorted, dest):
    mesh = plsc.VectorSubcoreMesh(core_axis_name="c", subcore_axis_name="s")

    @functools.partial(
        pl.kernel, mesh=mesh,
        out_type=jax.ShapeDtypeStruct((T, DM), jnp.float32),
        scratch_types=[
            pltpu.VMEM((TPW, DM), jnp.float32),
            pltpu.VMEM((TPW, DM), jnp.float32),
            pltpu.VMEM((TPW,), jnp.int32),
            pltpu.SemaphoreType.DMA,
        ],
    )
    def comb(y_hbm, d_hbm, out_hbm, rows0_v, rows1_v, idx_v, sem):
        wid = jax.lax.axis_index("s") * 2 + jax.lax.axis_index("c")
        base = wid * TPW
        pltpu.sync_copy(d_hbm.at[pl.ds(base, TPW)], idx_v)
        pltpu.async_copy(y_hbm.at[idx_v], rows0_v, sem).wait()
        pltpu.sync_copy(d_hbm.at[pl.ds(T + base, TPW)], idx_v)
        pltpu.async_copy(y_hbm.at[idx_v], rows1_v, sem).wait()

        def row_body(t, carry):
            def vec_body(v, carry2):
                sl = pl.ds(v * 16, 16)
                rows0_v[t, sl] = rows0_v[t, sl] + rows1_v[t, sl]
                return carry2
            return jax.lax.fori_loop(0, DM // 16, vec_body, carry)

        jax.lax.fori_loop(0, TPW, row_body, 0)
        pltpu.sync_copy(rows0_v, out_hbm.at[pl.ds(base, TPW)])

    return comb(y_sorted, dest)


def kernel(hidden_states, gate_weight, gate_up_proj, gate_up_bias, down_proj,
           down_bias):
    logits, dest, wall, sarr = _router(hidden_states, gate_weight.T)
    dest1 = dest.reshape(2 * T)
    wall1 = wall.reshape(2 * T)
    s1 = sarr.reshape(NS + 8)
    gub = gate_up_bias.reshape(E, 1, 2 * DF)
    db = down_bias.reshape(E, 1, DM)
    x_sorted, w_sorted = _dispatch(hidden_states, dest1, wall1)
    y_sorted = _gmm(s1, x_sorted, gate_up_proj, gub, down_proj, db,
                    w_sorted.reshape(NPAD, 1))
    out = _combine(y_sorted, dest1)
    return out, logits


# speedup vs baseline: 1.4240x; 1.4240x over previous
import functools

import jax
import jax.numpy as jnp
from jax.experimental import pallas as pl
from jax.experimental.pallas import tpu as pltpu
from jax.experimental.pallas import tpu_sc as plsc

E = 8
DM = 768
DF = 2048
T = 2048
BLK = 256
NPAD = 2 * T + E * BLK
NBLK = NPAD // BLK
NS = 32
NW = 32
TPW = T // NW


def _router_body(h_ref, gwt_ref, logits_ref, dest_ref, wall_ref, sarr_ref):
    hb = h_ref[...].astype(jnp.bfloat16)
    logits = jax.lax.dot_general(
        hb, gwt_ref[...].astype(jnp.bfloat16), (((1,), (0,)), ((), ())),
        preferred_element_type=jnp.float32)
    logits_ref[...] = logits
    m = jnp.max(logits, axis=1, keepdims=True)
    ex = jnp.exp(logits - m)
    p = ex / jnp.sum(ex, axis=1, keepdims=True)
    idxs = jax.lax.broadcasted_iota(jnp.int32, (T, E), 1)
    m0 = jnp.max(p, axis=1, keepdims=True)
    i0 = jnp.min(jnp.where(p == m0, idxs, E), axis=1, keepdims=True)
    p1 = jnp.where(idxs == i0, -jnp.inf, p)
    m1 = jnp.max(p1, axis=1, keepdims=True)
    i1 = jnp.min(jnp.where(p1 == m1, idxs, E), axis=1, keepdims=True)
    s = m0 + m1
    wall_ref[...] = jnp.concatenate([m0 / s, m1 / s], axis=0)

    ohc = jnp.concatenate([idxs == i0, idxs == i1], axis=0)
    c = ohc.astype(jnp.float32)
    sh = 1
    while sh < 2 * T:
        c = c + jnp.concatenate(
            [jnp.zeros((sh, E), jnp.float32), c[:2 * T - sh]], axis=0)
        sh *= 2
    counts = c[2 * T - 1:2 * T, :].astype(jnp.int32)
    padded = ((counts + (BLK - 1)) // BLK) * BLK
    incl = padded
    sh = 1
    while sh < E:
        incl = incl + jnp.concatenate(
            [jnp.zeros((1, sh), jnp.int32), incl[:, :E - sh]], axis=1)
        sh *= 2
    off = incl - padded
    dest = jnp.sum(
        jnp.where(ohc, c + off.astype(jnp.float32) - 1.0, 0.0),
        axis=1, keepdims=True)
    dest_ref[...] = dest.astype(jnp.int32)

    ub = incl[0:1, E - 1:E] // BLK
    bs = jax.lax.broadcasted_iota(jnp.int32, (NS, 1), 0) * BLK
    be = jnp.sum((incl <= bs).astype(jnp.int32), axis=1, keepdims=True)
    eidx = jax.lax.broadcasted_iota(jnp.int32, (1, E), 1)
    last_e = jnp.max(jnp.where(padded > 0, eidx, 0), axis=1, keepdims=True)
    be = jnp.where(bs < ub * BLK, be, last_e)
    sarr_ref[...] = jnp.concatenate(
        [be, jnp.broadcast_to(ub, (8, 1))], axis=0)


def _router(hidden_states, gwt):
    return pl.pallas_call(
        _router_body,
        out_shape=[
            jax.ShapeDtypeStruct((T, E), jnp.float32),
            jax.ShapeDtypeStruct((2 * T, 1), jnp.int32),
            jax.ShapeDtypeStruct((2 * T, 1), jnp.float32),
            jax.ShapeDtypeStruct((NS + 8, 1), jnp.int32),
        ],
    )(hidden_states, gwt)


def _dispatch(hidden_states, dest):
    mesh = plsc.VectorSubcoreMesh(core_axis_name="c", subcore_axis_name="s")

    @functools.partial(
        pl.kernel, mesh=mesh,
        out_type=jax.ShapeDtypeStruct((NPAD, DM), jnp.float32),
        scratch_types=[
            pltpu.VMEM((TPW, DM), jnp.float32),
            pltpu.VMEM((TPW,), jnp.int32),
            pltpu.SemaphoreType.DMA,
        ],
    )
    def disp(h_hbm, d_hbm, x_out, rows_v, idx_v, sem):
        wid = jax.lax.axis_index("s") * 2 + jax.lax.axis_index("c")
        base = wid * TPW
        pltpu.sync_copy(h_hbm.at[pl.ds(base, TPW)], rows_v)
        pltpu.sync_copy(d_hbm.at[pl.ds(base, TPW)], idx_v)
        pltpu.async_copy(rows_v, x_out.at[idx_v], sem).wait()
        pltpu.sync_copy(d_hbm.at[pl.ds(T + base, TPW)], idx_v)
        pltpu.async_copy(rows_v, x_out.at[idx_v], sem).wait()

    return disp(hidden_states, dest)


def _gmm_body(s_ref, x_ref, wg_ref, wu_ref, bg_ref, bu_ref, wd_ref, bd_ref,
              y_ref):
    @pl.when(pl.program_id(0) < s_ref[NS])
    def _():
        x = x_ref[...]
        g = jnp.dot(x, wg_ref[0], precision=jax.lax.Precision.DEFAULT,
                    preferred_element_type=jnp.float32) + bg_ref[0]
        u = jnp.dot(x, wu_ref[0], precision=jax.lax.Precision.DEFAULT,
                    preferred_element_type=jnp.float32) + bu_ref[0]
        inter = (g * jax.lax.logistic(g)) * u
        y_ref[...] = jnp.dot(inter, wd_ref[0],
                             precision=jax.lax.Precision.DEFAULT,
                             preferred_element_type=jnp.float32) + bd_ref[0]


def _gmm(s1, x_sorted, gate_up_proj, gub, down_proj, db):
    grid_spec = pltpu.PrefetchScalarGridSpec(
        num_scalar_prefetch=1,
        grid=(NBLK,),
        in_specs=[
            pl.BlockSpec((BLK, DM), lambda i, s: (jnp.minimum(i, s[NS] - 1), 0)),
            pl.BlockSpec((1, DM, DF), lambda i, s: (s[i], 0, 0)),
            pl.BlockSpec((1, DM, DF), lambda i, s: (s[i], 0, 1)),
            pl.BlockSpec((1, 1, DF), lambda i, s: (s[i], 0, 0)),
            pl.BlockSpec((1, 1, DF), lambda i, s: (s[i], 0, 1)),
            pl.BlockSpec((1, DF, DM), lambda i, s: (s[i], 0, 0)),
            pl.BlockSpec((1, 1, DM), lambda i, s: (s[i], 0, 0)),
        ],
        out_specs=pl.BlockSpec((BLK, DM),
                               lambda i, s: (jnp.minimum(i, s[NS] - 1), 0)),
        scratch_shapes=[],
    )
    return pl.pallas_call(
        _gmm_body,
        grid_spec=grid_spec,
        out_shape=jax.ShapeDtypeStruct((NPAD, DM), jnp.float32),
    )(s1, x_sorted, gate_up_proj, gate_up_proj, gub, gub, down_proj, db)


def _combine(y_sorted, dest):
    mesh = plsc.VectorSubcoreMesh(core_axis_name="c", subcore_axis_name="s")

    @functools.partial(
        pl.kernel, mesh=mesh,
        out_type=[
            jax.ShapeDtypeStruct((T, DM), jnp.float32),
            jax.ShapeDtypeStruct((T, DM), jnp.float32),
        ],
        scratch_types=[
            pltpu.VMEM((TPW, DM), jnp.float32),
            pltpu.VMEM((TPW,), jnp.int32),
            pltpu.SemaphoreType.DMA,
        ],
    )
    def comb(y_hbm, d_hbm, y0_out, y1_out, rows_v, idx_v, sem):
        wid = jax.lax.axis_index("s") * 2 + jax.lax.axis_index("c")
        base = wid * TPW
        pltpu.sync_copy(d_hbm.at[pl.ds(base, TPW)], idx_v)
        pltpu.async_copy(y_hbm.at[idx_v], rows_v, sem).wait()
        pltpu.sync_copy(rows_v, y0_out.at[pl.ds(base, TPW)])
        pltpu.sync_copy(d_hbm.at[pl.ds(T + base, TPW)], idx_v)
        pltpu.async_copy(y_hbm.at[idx_v], rows_v, sem).wait()
        pltpu.sync_copy(rows_v, y1_out.at[pl.ds(base, TPW)])

    return comb(y_sorted, dest)


def _mix_body(y0_ref, y1_ref, w0_ref, w1_ref, out_ref):
    out_ref[...] = y0_ref[...] * w0_ref[...] + y1_ref[...] * w1_ref[...]


def _mix(y0, y1, w0, w1):
    return pl.pallas_call(
        _mix_body,
        out_shape=jax.ShapeDtypeStruct((T, DM), jnp.float32),
    )(y0, y1, w0, w1)


def kernel(hidden_states, gate_weight, gate_up_proj, gate_up_bias, down_proj,
           down_bias):
    logits, dest, wall, sarr = _router(hidden_states, gate_weight.T)
    dest1 = dest.reshape(2 * T)
    s1 = sarr.reshape(NS + 8)
    gub = gate_up_bias.reshape(E, 1, 2 * DF)
    db = down_bias.reshape(E, 1, DM)
    x_sorted = _dispatch(hidden_states, dest1)
    y_sorted = _gmm(s1, x_sorted, gate_up_proj, gub, down_proj, db)
    y0, y1 = _combine(y_sorted, dest1)
    out = _mix(y0, y1, wall[:T], wall[T:])
    return out, logits


# speedup vs baseline: 1.5142x; 1.0634x over previous
import functools

import jax
import jax.numpy as jnp
from jax.experimental import pallas as pl
from jax.experimental.pallas import tpu as pltpu
from jax.experimental.pallas import tpu_sc as plsc

E = 8
DM = 768
DF = 2048
T = 2048
BLK = 256
NPAD = 2 * T + E * BLK
NBLK = NPAD // BLK
NS = 32
NW = 32
TPW = T // NW


DH = DM // 2


def _pack32(x_bf):
    u = jax.lax.bitcast_convert_type(x_bf, jnp.uint16)
    lo = u[..., :DH].astype(jnp.uint32)
    hi = u[..., DH:].astype(jnp.uint32)
    return (lo | (hi << 16)).astype(jnp.int32)


def _unpack32(x32):
    u = x32.astype(jnp.uint32)
    lo = jax.lax.bitcast_convert_type((u & 0xFFFF).astype(jnp.uint16),
                                      jnp.bfloat16)
    hi = jax.lax.bitcast_convert_type((u >> 16).astype(jnp.uint16),
                                      jnp.bfloat16)
    return jnp.concatenate([lo, hi], axis=-1).astype(jnp.float32)


def _router_body(h_ref, gwt_ref, logits_ref, dest_ref, wall_ref, sarr_ref,
                 hb_ref):
    hb = h_ref[...].astype(jnp.bfloat16)
    hb_ref[...] = _pack32(hb)
    logits = jax.lax.dot_general(
        hb, gwt_ref[...].astype(jnp.bfloat16), (((1,), (0,)), ((), ())),
        preferred_element_type=jnp.float32)
    logits_ref[...] = logits
    m = jnp.max(logits, axis=1, keepdims=True)
    ex = jnp.exp(logits - m)
    p = ex / jnp.sum(ex, axis=1, keepdims=True)
    idxs = jax.lax.broadcasted_iota(jnp.int32, (T, E), 1)
    m0 = jnp.max(p, axis=1, keepdims=True)
    i0 = jnp.min(jnp.where(p == m0, idxs, E), axis=1, keepdims=True)
    p1 = jnp.where(idxs == i0, -jnp.inf, p)
    m1 = jnp.max(p1, axis=1, keepdims=True)
    i1 = jnp.min(jnp.where(p1 == m1, idxs, E), axis=1, keepdims=True)
    s = m0 + m1
    wall_ref[...] = jnp.concatenate([m0 / s, m1 / s], axis=0)

    ohc = jnp.concatenate([idxs == i0, idxs == i1], axis=0)
    c = ohc.astype(jnp.float32)
    sh = 1
    while sh < 2 * T:
        c = c + jnp.concatenate(
            [jnp.zeros((sh, E), jnp.float32), c[:2 * T - sh]], axis=0)
        sh *= 2
    counts = c[2 * T - 1:2 * T, :].astype(jnp.int32)
    padded = ((counts + (BLK - 1)) // BLK) * BLK
    incl = padded
    sh = 1
    while sh < E:
        incl = incl + jnp.concatenate(
            [jnp.zeros((1, sh), jnp.int32), incl[:, :E - sh]], axis=1)
        sh *= 2
    off = incl - padded
    dest = jnp.sum(
        jnp.where(ohc, c + off.astype(jnp.float32) - 1.0, 0.0),
        axis=1, keepdims=True)
    dest_ref[...] = dest.astype(jnp.int32)

    ub = incl[0:1, E - 1:E] // BLK
    bs = jax.lax.broadcasted_iota(jnp.int32, (NS, 1), 0) * BLK
    be = jnp.sum((incl <= bs).astype(jnp.int32), axis=1, keepdims=True)
    eidx = jax.lax.broadcasted_iota(jnp.int32, (1, E), 1)
    last_e = jnp.max(jnp.where(padded > 0, eidx, 0), axis=1, keepdims=True)
    be = jnp.where(bs < ub * BLK, be, last_e)
    sarr_ref[...] = jnp.concatenate(
        [be, jnp.broadcast_to(ub, (8, 1))], axis=0)


def _router(hidden_states, gwt):
    return pl.pallas_call(
        _router_body,
        out_shape=[
            jax.ShapeDtypeStruct((T, E), jnp.float32),
            jax.ShapeDtypeStruct((2 * T, 1), jnp.int32),
            jax.ShapeDtypeStruct((2 * T, 1), jnp.float32),
            jax.ShapeDtypeStruct((NS + 8, 1), jnp.int32),
            jax.ShapeDtypeStruct((T, DH), jnp.int32),
        ],
    )(hidden_states, gwt)


def _dispatch(hidden_states, dest):
    mesh = plsc.VectorSubcoreMesh(core_axis_name="c", subcore_axis_name="s")

    @functools.partial(
        pl.kernel, mesh=mesh,
        out_type=jax.ShapeDtypeStruct((NPAD, DH), jnp.int32),
        scratch_types=[
            pltpu.VMEM((TPW, DH), jnp.int32),
            pltpu.VMEM((TPW,), jnp.int32),
            pltpu.SemaphoreType.DMA,
        ],
    )
    def disp(h_hbm, d_hbm, x_out, rows_v, idx_v, sem):
        wid = jax.lax.axis_index("s") * 2 + jax.lax.axis_index("c")
        base = wid * TPW
        pltpu.sync_copy(h_hbm.at[pl.ds(base, TPW)], rows_v)
        pltpu.sync_copy(d_hbm.at[pl.ds(base, TPW)], idx_v)
        pltpu.async_copy(rows_v, x_out.at[idx_v], sem).wait()
        pltpu.sync_copy(d_hbm.at[pl.ds(T + base, TPW)], idx_v)
        pltpu.async_copy(rows_v, x_out.at[idx_v], sem).wait()

    return disp(hidden_states, dest)


def _gmm_body(s_ref, x_ref, wg_ref, wu_ref, bg_ref, bu_ref, wd_ref, bd_ref,
              y_ref):
    @pl.when(pl.program_id(0) < s_ref[NS])
    def _():
        x = _unpack32(x_ref[...])
        g = jnp.dot(x, wg_ref[0], precision=jax.lax.Precision.DEFAULT,
                    preferred_element_type=jnp.float32) + bg_ref[0]
        u = jnp.dot(x, wu_ref[0], precision=jax.lax.Precision.DEFAULT,
                    preferred_element_type=jnp.float32) + bu_ref[0]
        inter = (g * jax.lax.logistic(g)) * u
        y = jnp.dot(inter, wd_ref[0], precision=jax.lax.Precision.DEFAULT,
                    preferred_element_type=jnp.float32) + bd_ref[0]
        y_ref[...] = _pack32(y.astype(jnp.bfloat16))


def _gmm(s1, x_sorted, gate_up_proj, gub, down_proj, db):
    grid_spec = pltpu.PrefetchScalarGridSpec(
        num_scalar_prefetch=1,
        grid=(NBLK,),
        in_specs=[
            pl.BlockSpec((BLK, DH), lambda i, s: (jnp.minimum(i, s[NS] - 1), 0)),
            pl.BlockSpec((1, DM, DF), lambda i, s: (s[i], 0, 0)),
            pl.BlockSpec((1, DM, DF), lambda i, s: (s[i], 0, 1)),
            pl.BlockSpec((1, 1, DF), lambda i, s: (s[i], 0, 0)),
            pl.BlockSpec((1, 1, DF), lambda i, s: (s[i], 0, 1)),
            pl.BlockSpec((1, DF, DM), lambda i, s: (s[i], 0, 0)),
            pl.BlockSpec((1, 1, DM), lambda i, s: (s[i], 0, 0)),
        ],
        out_specs=pl.BlockSpec((BLK, DH),
                               lambda i, s: (jnp.minimum(i, s[NS] - 1), 0)),
        scratch_shapes=[],
    )
    return pl.pallas_call(
        _gmm_body,
        grid_spec=grid_spec,
        out_shape=jax.ShapeDtypeStruct((NPAD, DH), jnp.int32),
    )(s1, x_sorted, gate_up_proj, gate_up_proj, gub, gub, down_proj, db)


def _combine(y_sorted, dest):
    mesh = plsc.VectorSubcoreMesh(core_axis_name="c", subcore_axis_name="s")

    @functools.partial(
        pl.kernel, mesh=mesh,
        out_type=[
            jax.ShapeDtypeStruct((T, DH), jnp.int32),
            jax.ShapeDtypeStruct((T, DH), jnp.int32),
        ],
        scratch_types=[
            pltpu.VMEM((TPW, DH), jnp.int32),
            pltpu.VMEM((TPW,), jnp.int32),
            pltpu.SemaphoreType.DMA,
        ],
    )
    def comb(y_hbm, d_hbm, y0_out, y1_out, rows_v, idx_v, sem):
        wid = jax.lax.axis_index("s") * 2 + jax.lax.axis_index("c")
        base = wid * TPW
        pltpu.sync_copy(d_hbm.at[pl.ds(base, TPW)], idx_v)
        pltpu.async_copy(y_hbm.at[idx_v], rows_v, sem).wait()
        pltpu.sync_copy(rows_v, y0_out.at[pl.ds(base, TPW)])
        pltpu.sync_copy(d_hbm.at[pl.ds(T + base, TPW)], idx_v)
        pltpu.async_copy(y_hbm.at[idx_v], rows_v, sem).wait()
        pltpu.sync_copy(rows_v, y1_out.at[pl.ds(base, TPW)])

    return comb(y_sorted, dest)


def _mix_body(y0_ref, y1_ref, w0_ref, w1_ref, out_ref):
    out_ref[...] = (_unpack32(y0_ref[...]) * w0_ref[...] +
                    _unpack32(y1_ref[...]) * w1_ref[...])


def _mix(y0, y1, w0, w1):
    return pl.pallas_call(
        _mix_body,
        out_shape=jax.ShapeDtypeStruct((T, DM), jnp.float32),
    )(y0, y1, w0, w1)


def kernel(hidden_states, gate_weight, gate_up_proj, gate_up_bias, down_proj,
           down_bias):
    logits, dest, wall, sarr, hb = _router(hidden_states, gate_weight.T)
    dest1 = dest.reshape(2 * T)
    s1 = sarr.reshape(NS + 8)
    gub = gate_up_bias.reshape(E, 1, 2 * DF)
    db = down_bias.reshape(E, 1, DM)
    x_sorted = _dispatch(hb, dest1)
    y_sorted = _gmm(s1, x_sorted, gate_up_proj, gub, down_proj, db)
    y0, y1 = _combine(y_sorted, dest1)
    out = _mix(y0, y1, wall[:T], wall[T:])
    return out, logits


# speedup vs baseline: 1.5524x; 1.0252x over previous
import functools

import jax
import jax.numpy as jnp
from jax.experimental import pallas as pl
from jax.experimental.pallas import tpu as pltpu
from jax.experimental.pallas import tpu_sc as plsc

E = 8
DM = 768
DF = 2048
T = 2048
BLK = 256
NPAD = 2 * T + E * BLK
NBLK = NPAD // BLK
NS = 32
NW = 32
TPW = T // NW


DH = DM // 2


def _pack32(x_bf):
    u = jax.lax.bitcast_convert_type(x_bf, jnp.uint16)
    lo = u[..., :DH].astype(jnp.uint32)
    hi = u[..., DH:].astype(jnp.uint32)
    return (lo | (hi << 16)).astype(jnp.int32)


def _unpack32(x32):
    u = x32.astype(jnp.uint32)
    lo = jax.lax.bitcast_convert_type((u & 0xFFFF).astype(jnp.uint16),
                                      jnp.bfloat16)
    hi = jax.lax.bitcast_convert_type((u >> 16).astype(jnp.uint16),
                                      jnp.bfloat16)
    return jnp.concatenate([lo, hi], axis=-1).astype(jnp.float32)


def _router_body(h_ref, gw_ref, logits_ref, dest_ref, w0_ref, w1_ref,
                 sarr_ref, hb_ref):
    hb = h_ref[...].astype(jnp.bfloat16)
    hb_ref[...] = _pack32(hb)
    logits = jax.lax.dot_general(
        hb, gw_ref[...].astype(jnp.bfloat16), (((1,), (1,)), ((), ())),
        preferred_element_type=jnp.float32)
    logits_ref[...] = logits
    m = jnp.max(logits, axis=1, keepdims=True)
    ex = jnp.exp(logits - m)
    p = ex / jnp.sum(ex, axis=1, keepdims=True)
    idxs = jax.lax.broadcasted_iota(jnp.int32, (T, E), 1)
    m0 = jnp.max(p, axis=1, keepdims=True)
    i0 = jnp.min(jnp.where(p == m0, idxs, E), axis=1, keepdims=True)
    p1 = jnp.where(idxs == i0, -jnp.inf, p)
    m1 = jnp.max(p1, axis=1, keepdims=True)
    i1 = jnp.min(jnp.where(p1 == m1, idxs, E), axis=1, keepdims=True)
    s = m0 + m1
    w0_ref[...] = m0 / s
    w1_ref[...] = m1 / s

    ohc = jnp.concatenate([idxs == i0, idxs == i1], axis=0)
    c = ohc.astype(jnp.float32)
    sh = 1
    while sh < 2 * T:
        c = c + jnp.concatenate(
            [jnp.zeros((sh, E), jnp.float32), c[:2 * T - sh]], axis=0)
        sh *= 2
    counts = c[2 * T - 1:2 * T, :].astype(jnp.int32)
    padded = ((counts + (BLK - 1)) // BLK) * BLK
    incl = padded
    sh = 1
    while sh < E:
        incl = incl + jnp.concatenate(
            [jnp.zeros((1, sh), jnp.int32), incl[:, :E - sh]], axis=1)
        sh *= 2
    off = incl - padded
    dest = jnp.sum(
        jnp.where(ohc, c + off.astype(jnp.float32) - 1.0, 0.0),
        axis=1, keepdims=True)
    dest_ref[...] = dest.astype(jnp.int32)

    ub = incl[0:1, E - 1:E] // BLK
    bs = jax.lax.broadcasted_iota(jnp.int32, (NS, 1), 0) * BLK
    be = jnp.sum((incl <= bs).astype(jnp.int32), axis=1, keepdims=True)
    eidx = jax.lax.broadcasted_iota(jnp.int32, (1, E), 1)
    last_e = jnp.max(jnp.where(padded > 0, eidx, 0), axis=1, keepdims=True)
    be = jnp.where(bs < ub * BLK, be, last_e)
    sarr_ref[...] = jnp.concatenate(
        [be, jnp.broadcast_to(ub, (8, 1))], axis=0)


def _router(hidden_states, gw):
    return pl.pallas_call(
        _router_body,
        out_shape=[
            jax.ShapeDtypeStruct((T, E), jnp.float32),
            jax.ShapeDtypeStruct((2 * T, 1), jnp.int32),
            jax.ShapeDtypeStruct((T, 1), jnp.float32),
            jax.ShapeDtypeStruct((T, 1), jnp.float32),
            jax.ShapeDtypeStruct((NS + 8, 1), jnp.int32),
            jax.ShapeDtypeStruct((T, DH), jnp.int32),
        ],
    )(hidden_states, gw)


def _dispatch(hidden_states, dest):
    mesh = plsc.VectorSubcoreMesh(core_axis_name="c", subcore_axis_name="s")

    @functools.partial(
        pl.kernel, mesh=mesh,
        out_type=jax.ShapeDtypeStruct((NPAD, DH), jnp.int32),
        scratch_types=[
            pltpu.VMEM((TPW, DH), jnp.int32),
            pltpu.VMEM((TPW,), jnp.int32),
            pltpu.SemaphoreType.DMA,
        ],
    )
    def disp(h_hbm, d_hbm, x_out, rows_v, idx_v, sem):
        wid = jax.lax.axis_index("s") * 2 + jax.lax.axis_index("c")
        base = wid * TPW
        pltpu.sync_copy(h_hbm.at[pl.ds(base, TPW)], rows_v)
        pltpu.sync_copy(d_hbm.at[pl.ds(base, TPW)], idx_v)
        pltpu.async_copy(rows_v, x_out.at[idx_v], sem).wait()
        pltpu.sync_copy(d_hbm.at[pl.ds(T + base, TPW)], idx_v)
        pltpu.async_copy(rows_v, x_out.at[idx_v], sem).wait()

    return disp(hidden_states, dest)


def _gmm_body(s_ref, x_ref, wg_ref, wu_ref, bg_ref, bu_ref, wd_ref, bd_ref,
              y_ref):
    @pl.when(pl.program_id(0) < s_ref[NS])
    def _():
        x = _unpack32(x_ref[...])
        g = jnp.dot(x, wg_ref[0], precision=jax.lax.Precision.DEFAULT,
                    preferred_element_type=jnp.float32) + bg_ref[0]
        u = jnp.dot(x, wu_ref[0], precision=jax.lax.Precision.DEFAULT,
                    preferred_element_type=jnp.float32) + bu_ref[0]
        inter = (g * jax.lax.logistic(g)) * u
        y = jnp.dot(inter, wd_ref[0], precision=jax.lax.Precision.DEFAULT,
                    preferred_element_type=jnp.float32) + bd_ref[0]
        y_ref[...] = _pack32(y.astype(jnp.bfloat16))


def _gmm(s1, x_sorted, gate_up_proj, gub, down_proj, db):
    grid_spec = pltpu.PrefetchScalarGridSpec(
        num_scalar_prefetch=1,
        grid=(NBLK,),
        in_specs=[
            pl.BlockSpec((BLK, DH), lambda i, s: (jnp.minimum(i, s[NS] - 1), 0)),
            pl.BlockSpec((1, DM, DF), lambda i, s: (s[i], 0, 0)),
            pl.BlockSpec((1, DM, DF), lambda i, s: (s[i], 0, 1)),
            pl.BlockSpec((1, 1, DF), lambda i, s: (s[i], 0, 0)),
            pl.BlockSpec((1, 1, DF), lambda i, s: (s[i], 0, 1)),
            pl.BlockSpec((1, DF, DM), lambda i, s: (s[i], 0, 0)),
            pl.BlockSpec((1, 1, DM), lambda i, s: (s[i], 0, 0)),
        ],
        out_specs=pl.BlockSpec((BLK, DH),
                               lambda i, s: (jnp.minimum(i, s[NS] - 1), 0)),
        scratch_shapes=[],
    )
    return pl.pallas_call(
        _gmm_body,
        grid_spec=grid_spec,
        out_shape=jax.ShapeDtypeStruct((NPAD, DH), jnp.int32),
    )(s1, x_sorted, gate_up_proj, gate_up_proj, gub, gub, down_proj, db)


def _combine(y_sorted, dest):
    mesh = plsc.VectorSubcoreMesh(core_axis_name="c", subcore_axis_name="s")

    @functools.partial(
        pl.kernel, mesh=mesh,
        out_type=[
            jax.ShapeDtypeStruct((T, DH), jnp.int32),
            jax.ShapeDtypeStruct((T, DH), jnp.int32),
        ],
        scratch_types=[
            pltpu.VMEM((TPW, DH), jnp.int32),
            pltpu.VMEM((TPW,), jnp.int32),
            pltpu.SemaphoreType.DMA,
        ],
    )
    def comb(y_hbm, d_hbm, y0_out, y1_out, rows_v, idx_v, sem):
        wid = jax.lax.axis_index("s") * 2 + jax.lax.axis_index("c")
        base = wid * TPW
        pltpu.sync_copy(d_hbm.at[pl.ds(base, TPW)], idx_v)
        pltpu.async_copy(y_hbm.at[idx_v], rows_v, sem).wait()
        pltpu.sync_copy(rows_v, y0_out.at[pl.ds(base, TPW)])
        pltpu.sync_copy(d_hbm.at[pl.ds(T + base, TPW)], idx_v)
        pltpu.async_copy(y_hbm.at[idx_v], rows_v, sem).wait()
        pltpu.sync_copy(rows_v, y1_out.at[pl.ds(base, TPW)])

    return comb(y_sorted, dest)


def _mix_body(y0_ref, y1_ref, w0_ref, w1_ref, out_ref):
    out_ref[...] = (_unpack32(y0_ref[...]) * w0_ref[...] +
                    _unpack32(y1_ref[...]) * w1_ref[...])


def _mix(y0, y1, w0, w1):
    return pl.pallas_call(
        _mix_body,
        out_shape=jax.ShapeDtypeStruct((T, DM), jnp.float32),
    )(y0, y1, w0, w1)


def kernel(hidden_states, gate_weight, gate_up_proj, gate_up_bias, down_proj,
           down_bias):
    logits, dest, w0, w1, sarr, hb = _router(hidden_states, gate_weight)
    dest1 = dest.reshape(2 * T)
    s1 = sarr.reshape(NS + 8)
    gub = gate_up_bias.reshape(E, 1, 2 * DF)
    db = down_bias.reshape(E, 1, DM)
    x_sorted = _dispatch(hb, dest1)
    y_sorted = _gmm(s1, x_sorted, gate_up_proj, gub, down_proj, db)
    y0, y1 = _combine(y_sorted, dest1)
    out = _mix(y0, y1, w0, w1)
    return out, logits
